# baseline (speedup 1.0000x reference)
import functools

import jax
import jax.numpy as jnp
from jax.experimental import pallas as pl
from jax.experimental.pallas import tpu as pltpu
from jax.experimental.pallas import tpu_sc as plsc

_B, _S, _F = 4, 256, 256
_L = 64
_K = 1024
_T = _B * _S

_HI = jax.lax.Precision.HIGHEST

_EPS = jax.random.normal(jax.random.key(42), (_B, _S, _L),
                         dtype=jnp.float32).reshape(_T, _L)

_TT = 256


def _encvq_body(x_ref, w1_ref, b1_ref, w2_ref, b2_ref, cbt_ref,
                eps_ref, mean_ref, logvar_ref, ze_ref, ids_ref):
    x = x_ref[...]
    h = jnp.maximum(
        jnp.dot(x, w1_ref[...], preferred_element_type=jnp.float32,
                precision=_HI) + b1_ref[...], 0.0)
    enc = jnp.dot(h, w2_ref[...], preferred_element_type=jnp.float32,
                  precision=_HI) + b2_ref[...]
    mean = enc[:, :_L]
    logvar = enc[:, _L:]
    z_e = mean + jnp.exp(0.5 * logvar) * eps_ref[...]

    cbt = cbt_ref[...]
    csq = jnp.sum(cbt * cbt, axis=0, keepdims=True)
    cross = jnp.dot(z_e, cbt, preferred_element_type=jnp.float32,
                    precision=_HI)
    dist = csq - 2.0 * cross
    m = jnp.min(dist, axis=1, keepdims=True)
    lane = jax.lax.broadcasted_iota(jnp.int32, (_TT, _K), 1)
    ids = jnp.min(jnp.where(dist <= m, lane, _K), axis=1, keepdims=True)

    mean_ref[...] = mean
    logvar_ref[...] = logvar
    ze_ref[...] = z_e
    ids_ref[...] = ids


_LP = 128


def _sc_gather(codebook_padded, ids1d):
    info = plsc.get_sparse_core_info()
    nw = info.num_cores * info.num_subcores
    bpw = _T // nw
    mesh = plsc.VectorSubcoreMesh(core_axis_name="c", subcore_axis_name="s")

    @functools.partial(
        pl.kernel, mesh=mesh,
        out_type=jax.ShapeDtypeStruct((_T, _LP), jnp.float32),
        scratch_types=[
            pltpu.VMEM((bpw,), jnp.int32),
            pltpu.VMEM((bpw, _LP), jnp.float32),
            pltpu.SemaphoreType.DMA,
        ],
    )
    def _k(cb_hbm, idx_hbm, out_hbm, idx_v, rows_v, sem):
        wid = jax.lax.axis_index("s") * info.num_cores + jax.lax.axis_index("c")
        base = wid * bpw
        pltpu.sync_copy(idx_hbm.at[pl.ds(base, bpw)], idx_v)
        pltpu.async_copy(cb_hbm.at[idx_v], rows_v, sem).wait()
        pltpu.sync_copy(rows_v, out_hbm.at[pl.ds(base, bpw)])

    return _k(codebook_padded, ids1d)


_K1 = 4
_N2 = 8


def _dec_body(flat_ref, ze_ref, w1_ref, b1_ref, w2_ref, b2_ref,
              out_ref, vq_ref, acc_ref, vqacc_ref):
    k = pl.program_id(0)

    @pl.when(k < _K1)
    def _dec1():
        fl = flat_ref[...]
        part = jnp.dot(fl.astype(jnp.bfloat16),
                       w1_ref[...].astype(jnp.bfloat16),
                       preferred_element_type=jnp.float32)
        d = fl - ze_ref[...]
        vqp = jnp.sum(jnp.sum(d * d, axis=1, keepdims=True),
                      axis=0, keepdims=True)

        @pl.when(k == 0)
        def _init():
            acc_ref[...] = part
            vqacc_ref[...] = vqp

        @pl.when(k > 0)
        def _acc():
            acc_ref[...] = acc_ref[...] + part
            vqacc_ref[...] = vqacc_ref[...] + vqp

        @pl.when(k == _K1 - 1)
        def _fin():
            acc_ref[...] = jnp.maximum(acc_ref[...] + b1_ref[...], 0.0)
            vq_ref[...] = vqacc_ref[...] / (_T * _L)

    @pl.when(k >= _K1)
    def _dec2():
        t = jnp.dot(acc_ref[...].astype(jnp.bfloat16),
                    w2_ref[...].astype(jnp.bfloat16),
                    preferred_element_type=jnp.float32) + b2_ref[...]
        out_ref[...] = jnp.maximum(t, 0.0) + jnp.log1p(jnp.exp(-jnp.abs(t)))


def kernel(x, enc_W1, enc_b1, enc_W2, enc_b2, codebook,
           dec_W1, dec_b1, dec_W2, dec_b2):
    x2 = x.reshape(_T, _F)

    nts = _T // _TT
    mean, logvar, z_e, ids = pl.pallas_call(
        _encvq_body,
        grid=(nts,),
        in_specs=[
            pl.BlockSpec((_TT, _F), lambda i: (i, 0)),
            pl.BlockSpec((_F, 512), lambda i: (0, 0)),
            pl.BlockSpec((1, 512), lambda i: (0, 0)),
            pl.BlockSpec((512, 2 * _L), lambda i: (0, 0)),
            pl.BlockSpec((1, 2 * _L), lambda i: (0, 0)),
            pl.BlockSpec((_L, _K), lambda i: (0, 0)),
            pl.BlockSpec((_TT, _L), lambda i: (i, 0)),
        ],
        out_specs=(
            pl.BlockSpec((_TT, _L), lambda i: (i, 0)),
            pl.BlockSpec((_TT, _L), lambda i: (i, 0)),
            pl.BlockSpec((_TT, _L), lambda i: (i, 0)),
            pl.BlockSpec((_TT, 1), lambda i: (i, 0)),
        ),
        out_shape=(
            jax.ShapeDtypeStruct((_T, _L), jnp.float32),
            jax.ShapeDtypeStruct((_T, _L), jnp.float32),
            jax.ShapeDtypeStruct((_T, _L), jnp.float32),
            jax.ShapeDtypeStruct((_T, 1), jnp.int32),
        ),
    )(x2, enc_W1, enc_b1.reshape(1, 512), enc_W2, enc_b2.reshape(1, 2 * _L),
      codebook.T, _EPS)

    cb_pad = jnp.pad(codebook, ((0, 0), (0, _LP - _L)))
    z_q = _sc_gather(cb_pad, ids.reshape(_T))[:, :_L]

    flat = z_q.reshape(_B, _S * _L)
    ze_flat = z_e.reshape(_B, _S * _L)

    kc = (_S * _L) // _K1
    nc = (_S * _F) // _N2
    rec, vq = pl.pallas_call(
        _dec_body,
        grid=(_K1 + _N2,),
        in_specs=[
            pl.BlockSpec((_B, kc), lambda k: (0, jnp.minimum(k, _K1 - 1))),
            pl.BlockSpec((_B, kc), lambda k: (0, jnp.minimum(k, _K1 - 1))),
            pl.BlockSpec((kc, 512), lambda k: (jnp.minimum(k, _K1 - 1), 0)),
            pl.BlockSpec((1, 512), lambda k: (0, 0)),
            pl.BlockSpec((512, nc), lambda k: (0, jnp.maximum(k - _K1, 0))),
            pl.BlockSpec((1, nc), lambda k: (0, jnp.maximum(k - _K1, 0))),
        ],
        out_specs=(
            pl.BlockSpec((_B, nc), lambda k: (0, jnp.maximum(k - _K1, 0))),
            pl.BlockSpec((1, 1), lambda k: (0, 0)),
        ),
        out_shape=(
            jax.ShapeDtypeStruct((_B, _S * _F), jnp.float32),
            jax.ShapeDtypeStruct((1, 1), jnp.float32),
        ),
        scratch_shapes=[pltpu.VMEM((_B, 512), jnp.float32),
                        pltpu.VMEM((1, 1), jnp.float32)],
    )(flat, ze_flat, dec_W1, dec_b1.reshape(1, 512),
      dec_W2, dec_b2.reshape(1, _S * _F))

    reconstructed = rec.reshape(_B, _S, _F)
    return (reconstructed,
            mean.reshape(_B, _S, _L),
            logvar.reshape(_B, _S, _L),
            vq[0, 0])


# speedup vs baseline: 1.3330x; 1.3330x over previous
import jax
import jax.numpy as jnp
from jax.experimental import pallas as pl
from jax.experimental.pallas import tpu as pltpu

_B, _S, _F = 4, 256, 256
_L = 64
_K = 1024
_T = _B * _S

_HI = jax.lax.Precision.HIGHEST

_EPS = jax.random.normal(jax.random.key(42), (_B, _S, _L),
                         dtype=jnp.float32).reshape(_T, _L)

_TT = 256
_NQ = 4
_QR = (_S * _L) // _NQ
_N2 = 16
_NC = (_S * _F) // _N2
_STEPS = 4 + _NQ + _N2


def _w1_copy(w1_hbm, buf, sem, q):
    return pltpu.make_async_copy(
        w1_hbm.at[pl.ds(q * _QR, _QR), :], buf, sem)


def _w2_copy(w2_hbm, buf, sem, j):
    return pltpu.make_async_copy(
        w2_hbm.at[:, pl.ds(j * _NC, _NC)], buf, sem)


def _body(x_ref, w1e_ref, b1e_ref, w2e_ref, b2e_ref, cbt_ref, cb_ref,
          eps_ref, dw1_hbm, db1_ref, dw2_hbm, db2_ref,
          mean_ref, logvar_ref, rec_ref, vq_ref,
          zq_scr, acc_ref, vqacc_ref, w1a, w1b, w2a, w2b, w1sem, w2sem):
    k = pl.program_id(0)

    @pl.when(k == 0)
    def _prime():
        _w1_copy(dw1_hbm, w1a, w1sem.at[0], 0).start()
        _w1_copy(dw1_hbm, w1b, w1sem.at[1], 1).start()
        _w2_copy(dw2_hbm, w2a, w2sem.at[0], 0).start()
        _w2_copy(dw2_hbm, w2b, w2sem.at[1], 1).start()

    @pl.when(k < 4)
    def _encvq():
        x = x_ref[...]
        h = jnp.maximum(
            jnp.dot(x, w1e_ref[...], preferred_element_type=jnp.float32,
                    precision=_HI) + b1e_ref[...], 0.0)
        enc = jnp.dot(h, w2e_ref[...], preferred_element_type=jnp.float32,
                      precision=_HI) + b2e_ref[...]
        mean = enc[:, :_L]
        logvar = enc[:, _L:]
        z_e = mean + jnp.exp(0.5 * logvar) * eps_ref[...]

        cbt = cbt_ref[...]
        csq = jnp.sum(cbt * cbt, axis=0, keepdims=True)
        cross = jnp.dot(z_e, cbt, preferred_element_type=jnp.float32,
                        precision=_HI)
        dist = csq - 2.0 * cross
        m = jnp.min(dist, axis=1, keepdims=True)
        lane = jax.lax.broadcasted_iota(jnp.int32, (_TT, _K), 1)
        ids = jnp.min(jnp.where(dist <= m, lane, _K), axis=1, keepdims=True)
        onehot = (lane == ids).astype(jnp.float32)
        z_q = jnp.dot(onehot, cb_ref[...], preferred_element_type=jnp.float32,
                      precision=_HI)

        mean_ref[...] = mean
        logvar_ref[...] = logvar
        zq_scr[k] = z_q
        d = z_e - z_q
        vqp = jnp.sum(jnp.sum(d * d, axis=1, keepdims=True),
                      axis=0, keepdims=True)

        @pl.when(k == 0)
        def _vq0():
            vqacc_ref[...] = vqp

        @pl.when(k > 0)
        def _vqn():
            vqacc_ref[...] = vqacc_ref[...] + vqp

    for qq in range(_NQ):
        @pl.when(k == 4 + qq)
        def _dec1(qq=qq):
            buf = w1a if qq % 2 == 0 else w1b
            sem = w1sem.at[qq % 2]
            _w1_copy(dw1_hbm, buf, sem, qq).wait()
            chunk = zq_scr[:, qq * 64:(qq + 1) * 64, :].astype(jnp.bfloat16)
            w1q = buf[...].astype(jnp.bfloat16)
            part = jnp.zeros((_B, 512), jnp.float32)
            for jj in range(64):
                part = part + jnp.dot(chunk[:, jj, :],
                                      w1q[jj * 64:(jj + 1) * 64, :],
                                      preferred_element_type=jnp.float32)
            if qq + 2 < _NQ:
                _w1_copy(dw1_hbm, buf, sem, qq + 2).start()

            @pl.when(k == 4)
            def _init():
                acc_ref[...] = part

            @pl.when(k > 4)
            def _acc():
                acc_ref[...] = acc_ref[...] + part

            @pl.when(k == 4 + _NQ - 1)
            def _fin():
                acc_ref[...] = jnp.maximum(acc_ref[...] + db1_ref[...], 0.0)
                vq_ref[...] = vqacc_ref[...] / (_T * _L)

    def _dec2(buf, sem):
        j = k - 8
        pltpu.make_async_copy(
            dw2_hbm.at[:, pl.ds(j * _NC, _NC)], buf, sem).wait()
        t = jnp.dot(acc_ref[...].astype(jnp.bfloat16),
                    buf[...].astype(jnp.bfloat16),
                    preferred_element_type=jnp.float32) + db2_ref[...]
        rec_ref[...] = jnp.maximum(t, 0.0) + jnp.log1p(jnp.exp(-jnp.abs(t)))

        @pl.when(j + 2 < _N2)
        def _next():
            pltpu.make_async_copy(
                dw2_hbm.at[:, pl.ds((j + 2) * _NC, _NC)], buf, sem).start()

    @pl.when((k >= 8) & ((k - 8) % 2 == 0))
    def _dec2_even():
        _dec2(w2a, w2sem.at[0])

    @pl.when((k >= 8) & ((k - 8) % 2 == 1))
    def _dec2_odd():
        _dec2(w2b, w2sem.at[1])


def kernel(x, enc_W1, enc_b1, enc_W2, enc_b2, codebook,
           dec_W1, dec_b1, dec_W2, dec_b2):
    x2 = x.reshape(_T, _F)

    mean, logvar, rec, vq = pl.pallas_call(
        _body,
        grid=(_STEPS,),
        in_specs=[
            pl.BlockSpec((_TT, _F), lambda k: (jnp.minimum(k, 3), 0)),
            pl.BlockSpec((_F, 512), lambda k: (0, 0)),
            pl.BlockSpec((1, 512), lambda k: (0, 0)),
            pl.BlockSpec((512, 2 * _L), lambda k: (0, 0)),
            pl.BlockSpec((1, 2 * _L), lambda k: (0, 0)),
            pl.BlockSpec((_L, _K), lambda k: (0, 0)),
            pl.BlockSpec((_K, _L), lambda k: (0, 0)),
            pl.BlockSpec((_TT, _L), lambda k: (jnp.minimum(k, 3), 0)),
            pl.BlockSpec(memory_space=pl.ANY),
            pl.BlockSpec((1, 512), lambda k: (0, 0)),
            pl.BlockSpec(memory_space=pl.ANY),
            pl.BlockSpec((1, _NC), lambda k: (0, jnp.maximum(k - 8, 0))),
        ],
        out_specs=(
            pl.BlockSpec((_TT, _L), lambda k: (jnp.minimum(k, 3), 0)),
            pl.BlockSpec((_TT, _L), lambda k: (jnp.minimum(k, 3), 0)),
            pl.BlockSpec((_B, _NC), lambda k: (0, jnp.maximum(k - 8, 0))),
            pl.BlockSpec((1, 1), lambda k: (0, 0)),
        ),
        out_shape=(
            jax.ShapeDtypeStruct((_T, _L), jnp.float32),
            jax.ShapeDtypeStruct((_T, _L), jnp.float32),
            jax.ShapeDtypeStruct((_B, _S * _F), jnp.float32),
            jax.ShapeDtypeStruct((1, 1), jnp.float32),
        ),
        scratch_shapes=[
            pltpu.VMEM((_B, _S, _L), jnp.float32),
            pltpu.VMEM((_B, 512), jnp.float32),
            pltpu.VMEM((1, 1), jnp.float32),
            pltpu.VMEM((_QR, 512), jnp.float32),
            pltpu.VMEM((_QR, 512), jnp.float32),
            pltpu.VMEM((512, _NC), jnp.float32),
            pltpu.VMEM((512, _NC), jnp.float32),
            pltpu.SemaphoreType.DMA((2,)),
            pltpu.SemaphoreType.DMA((2,)),
        ],
    )(x2, enc_W1, enc_b1.reshape(1, 512), enc_W2, enc_b2.reshape(1, 2 * _L),
      codebook.T, codebook, _EPS, dec_W1, dec_b1.reshape(1, 512),
      dec_W2, dec_b2.reshape(1, _S * _F))

    return (rec.reshape(_B, _S, _F),
            mean.reshape(_B, _S, _L),
            logvar.reshape(_B, _S, _L),
            vq[0, 0])
